# Initial kernel scaffold; baseline (speedup 1.0000x reference)
import jax
import jax.numpy as jnp
from jax.experimental import pallas as pl


def kernel(f_rad, f_histo, rad_mask, histo_mask, W, b, token):
    raise NotImplementedError("write your pallas kernel here")


# speedup vs baseline: 560.8836x; 560.8836x over previous
import functools

import jax
import jax.numpy as jnp
from jax.experimental import pallas as pl

B, C, OUT = 1024, 256, 256
K = 16
EPS = 1e-12
BLK = 256


def _topk_sums(x):
    neg_inf = jnp.float32(-3.4e38)
    pos_inf = jnp.float32(3.4e38)

    def extract(v, sentinel, reducer):
        total = jnp.zeros((v.shape[0], 1), jnp.float32)
        remaining = jnp.full((v.shape[0], 1), jnp.float32(K))
        for _ in range(K):
            m = reducer(v, axis=1, keepdims=True)
            eq = v == m
            cnt = jnp.sum(eq.astype(jnp.float32), axis=1, keepdims=True)
            take = jnp.minimum(cnt, remaining)
            total = total + take * m
            remaining = remaining - take
            v = jnp.where(eq, sentinel, v)
        return total

    top = extract(x, neg_inf, jnp.max)
    bot = extract(x, pos_inf, jnp.min)
    return top, bot


def _body(f_rad, f_histo, radf, histof, wt, b, token, out_ref):
    fr = f_rad[...]
    fh = f_histo[...]
    sq_r = jnp.sum(fr * fr, axis=1, keepdims=True)
    nr = fr / jnp.maximum(jnp.sqrt(sq_r), EPS)
    sq_h = jnp.sum(fh * fh, axis=1, keepdims=True)
    denom = jnp.float32(K) * jnp.maximum(jnp.sqrt(sq_h), EPS)
    topsum, botsum = _topk_sums(fh)
    tp = topsum / denom
    bt = botsum / denom
    p = jnp.maximum(nr, 0.0)
    m = jnp.minimum(nr, 0.0)
    pos = tp * p + bt * m
    neg = -(bt * p + tp * m)
    flat = jnp.concatenate([pos, neg], axis=1)
    acc = jax.lax.dot_general(
        flat, wt[...],
        (((1,), (0,)), ((), ())),
        preferred_element_type=jnp.float32,
        precision=jax.lax.Precision.HIGHEST,
    )
    flag = 1.0 - radf[...] * histof[...]
    out_ref[...] = acc + b[...] + token[...] * flag


@jax.jit
def kernel(f_rad, f_histo, rad_mask, histo_mask, W, b, token):
    wt = W.T
    radf = rad_mask.astype(jnp.float32).reshape(B, 1)
    histof = histo_mask.astype(jnp.float32).reshape(B, 1)
    b2 = b.reshape(1, OUT)
    grid = (B // BLK,)
    return pl.pallas_call(
        _body,
        grid=grid,
        in_specs=[
            pl.BlockSpec((BLK, C), lambda i: (i, 0)),
            pl.BlockSpec((BLK, C), lambda i: (i, 0)),
            pl.BlockSpec((BLK, 1), lambda i: (i, 0)),
            pl.BlockSpec((BLK, 1), lambda i: (i, 0)),
            pl.BlockSpec((2 * C, OUT), lambda i: (0, 0)),
            pl.BlockSpec((1, OUT), lambda i: (0, 0)),
            pl.BlockSpec((1, OUT), lambda i: (0, 0)),
        ],
        out_specs=pl.BlockSpec((BLK, OUT), lambda i: (i, 0)),
        out_shape=jax.ShapeDtypeStruct((B, OUT), jnp.float32),
    )(f_rad, f_histo, radf, histof, wt, b2, token)
